# baseline (speedup 1.0000x reference)
import functools

import jax
import jax.numpy as jnp
from jax import lax
from jax.experimental import pallas as pl
from jax.experimental.pallas import tpu as pltpu
from jax.experimental.pallas import tpu_sc as plsc

S, B, N, H, T = 4, 8, 1024, 768, 4096

NC, NS, L = 2, 16, 16
NW = NC * NS
PER_W = (B * T) // NW
CHUNK = 64
NCHUNK = PER_W // CHUNK
QUARTERS = T // PER_W


def _cls_body(cls_ref, w_ref, b_ref, out_ref):
    out_ref[...] = lax.dot_general(
        cls_ref[...], w_ref[...], (((1,), (1,)), ((), ())),
        preferred_element_type=jnp.float32) + b_ref[...]


def _class_project(cls2, W, b2):
    return pl.pallas_call(
        _cls_body,
        out_shape=jax.ShapeDtypeStruct((B, H), jnp.float32),
    )(cls2, W, b2)


def _sc_gather(table, img_flat, tok_flat, cls_rows):
    mesh = plsc.VectorSubcoreMesh(
        core_axis_name="c", subcore_axis_name="s",
        num_cores=NC, num_subcores=NS)

    @functools.partial(
        pl.kernel,
        out_type=jax.ShapeDtypeStruct((B * (T + 1), H), jnp.float32),
        mesh=mesh,
        compiler_params=pltpu.CompilerParams(use_tc_tiling_on_sc=False),
        scratch_types=[
            pltpu.VMEM((PER_W,), jnp.int32),
            pltpu.VMEM((PER_W,), jnp.int32),
            pltpu.VMEM((PER_W,), jnp.int32),
            pltpu.VMEM((CHUNK, H), jnp.float32),
            pltpu.VMEM((CHUNK, H), jnp.float32),
            pltpu.VMEM((1, H), jnp.float32),
            pltpu.SemaphoreType.DMA,
            pltpu.SemaphoreType.DMA,
        ],
    )
    def k(table_hbm, img_hbm, tok_hbm, cls_hbm, out_hbm,
          img_v, tok_v, idx_v, buf0, buf1, cls_buf, sem0, sem1):
        wid = lax.axis_index("s") * NC + lax.axis_index("c")
        b_id = wid // QUARTERS
        q = wid % QUARTERS
        src_base = wid * PER_W
        dst_base = b_id * (T + 1) + 1 + q * PER_W

        pltpu.sync_copy(img_hbm.at[pl.ds(src_base, PER_W)], img_v)
        pltpu.sync_copy(tok_hbm.at[pl.ds(src_base, PER_W)], tok_v)

        row_off = b_id * N

        def compute_idx(j, carry):
            sl = pl.ds(j * L, L)
            idx_v[sl] = img_v[sl] * (B * N) + (tok_v[sl] + row_off)
            return carry

        lax.fori_loop(0, PER_W // L, compute_idx, 0)

        @pl.when(q == 0)
        def _():
            pltpu.sync_copy(cls_hbm.at[pl.ds(b_id, 1)], cls_buf)
            pltpu.sync_copy(cls_buf, out_hbm.at[pl.ds(b_id * (T + 1), 1)])

        bufs = (buf0, buf1)
        sems = (sem0, sem1)
        prev = pltpu.async_copy(
            table_hbm.at[idx_v.at[pl.ds(0, CHUNK)]], buf0, sem0)
        for c in range(1, NCHUNK):
            cur = pltpu.async_copy(
                table_hbm.at[idx_v.at[pl.ds(c * CHUNK, CHUNK)]],
                bufs[c % 2], sems[c % 2])
            prev.wait()
            pltpu.sync_copy(bufs[(c - 1) % 2],
                            out_hbm.at[pl.ds(dst_base + (c - 1) * CHUNK, CHUNK)])
            prev = cur
        prev.wait()
        pltpu.sync_copy(bufs[(NCHUNK - 1) % 2],
                        out_hbm.at[pl.ds(dst_base + (NCHUNK - 1) * CHUNK, CHUNK)])

    return k(table, img_flat, tok_flat, cls_rows)


def kernel(class_tokens, patch_tokens, src_img, src_tok, W, b):
    Bn = src_img.shape[0]
    cls2 = jnp.transpose(class_tokens, (1, 2, 0, 3)).reshape(Bn, S * H)
    cls_rows = _class_project(cls2, W, b.reshape(1, H))

    table = patch_tokens.reshape(S * B * N, H)
    img_flat = src_img.astype(jnp.int32).reshape(B * T)
    tok_flat = src_tok.astype(jnp.int32).reshape(B * T)

    out = _sc_gather(table, img_flat, tok_flat, cls_rows)
    return out.reshape(Bn, T + 1, H)


# speedup vs baseline: 4.0699x; 4.0699x over previous
import functools

import jax
import jax.numpy as jnp
from jax import lax
from jax.experimental import pallas as pl
from jax.experimental.pallas import tpu as pltpu
from jax.experimental.pallas import tpu_sc as plsc

S, B, N, H, T = 4, 8, 1024, 768, 4096

NC, NS, L = 2, 16, 16
NW = NC * NS
PER_W = (B * T) // NW
CHUNK = 64
NCHUNK = PER_W // CHUNK
QUARTERS = T // PER_W


def _cls_body(cls_ref, w_ref, b_ref, out_ref):
    m = lax.dot_general(
        cls_ref[...], w_ref[...], (((1,), (1,)), ((), ())),
        preferred_element_type=jnp.float32) + b_ref[...]
    expanded = jnp.broadcast_to(m[:, None, :], (B, 8, H)).reshape(8 * B, H)
    rows = lax.broadcasted_iota(jnp.int32, (8 * B, H), 0)
    out_ref[...] = jnp.where(rows % 8 == 0, expanded, 0.0)


def _class_project(cls2, W, b2):
    return pl.pallas_call(
        _cls_body,
        out_shape=jax.ShapeDtypeStruct((8 * B, H), jnp.float32),
    )(cls2, W, b2)


def _sc_gather(table, img_flat, tok_flat, cls_rows):
    mesh = plsc.VectorSubcoreMesh(
        core_axis_name="c", subcore_axis_name="s",
        num_cores=NC, num_subcores=NS)

    @functools.partial(
        pl.kernel,
        out_type=jax.ShapeDtypeStruct((B, T + 1, H), jnp.float32),
        mesh=mesh,
        compiler_params=pltpu.CompilerParams(needs_layout_passes=False),
        scratch_types=[
            pltpu.VMEM((PER_W,), jnp.int32),
            pltpu.VMEM((PER_W,), jnp.int32),
            pltpu.VMEM((PER_W + 16,), jnp.int32),
            pltpu.VMEM((16,), jnp.int32),
            pltpu.VMEM((16,), jnp.int32),
            pltpu.VMEM((8,), jnp.int32),
            pltpu.VMEM((CHUNK, H), jnp.float32),
            pltpu.VMEM((CHUNK, H), jnp.float32),
            pltpu.VMEM((1, H), jnp.float32),
            pltpu.VMEM((8, H), jnp.float32),
            pltpu.SemaphoreType.DMA,
            pltpu.SemaphoreType.DMA,
        ],
    )
    def k(table_hbm, img_hbm, tok_hbm, cls_hbm, out_hbm,
          img_v, tok_v, idx_v, bimg_v, btok_v, tidx_v,
          buf0, buf1, row_buf, cls_buf, sem0, sem1):
        wid = lax.axis_index("s") * NC + lax.axis_index("c")
        b_id = wid // QUARTERS
        q = wid % QUARTERS
        src_base = wid * PER_W

        pltpu.sync_copy(img_hbm.at[pl.ds(src_base, PER_W)], img_v)
        pltpu.sync_copy(tok_hbm.at[pl.ds(src_base, PER_W)], tok_v)

        row_off = b_id * N
        lane = lax.iota(jnp.int32, 16)

        idx_v[pl.ds(0, 16)] = jnp.zeros((16,), jnp.int32)

        def compute_idx(j, carry):
            sl = pl.ds(j * L, L)
            g = img_v[sl] * (B * N) + (tok_v[sl] + row_off)
            idx_v[pl.ds(j * L + 1, L)] = g
            return carry

        lax.fori_loop(0, PER_W // L, compute_idx, 0)

        @pl.when(q > 0)
        def _():
            pltpu.sync_copy(img_hbm.at[pl.ds(src_base - 16, 16)], bimg_v)
            pltpu.sync_copy(tok_hbm.at[pl.ds(src_base - 16, 16)], btok_v)
            g = bimg_v[...] * (B * N) + (btok_v[...] + row_off)
            plsc.store_scatter(idx_v, [jnp.zeros((16,), jnp.int32)], g,
                               mask=lane == 15)

        bufs = (buf0, buf1)
        sems = (sem0, sem1)
        dst0 = q * PER_W
        prev = pltpu.async_copy(
            table_hbm.at[idx_v.at[pl.ds(0, CHUNK)]], buf0, sem0)
        for c in range(1, NCHUNK):
            cur = pltpu.async_copy(
                table_hbm.at[idx_v.at[pl.ds(c * CHUNK, CHUNK)]],
                bufs[c % 2], sems[c % 2])
            prev.wait()
            pltpu.sync_copy(
                bufs[(c - 1) % 2],
                out_hbm.at[b_id, pl.ds(dst0 + (c - 1) * CHUNK, CHUNK), :])
            prev = cur
        prev.wait()
        pltpu.sync_copy(
            bufs[(NCHUNK - 1) % 2],
            out_hbm.at[b_id, pl.ds(dst0 + (NCHUNK - 1) * CHUNK, CHUNK), :])

        @pl.when(q == 0)
        def _():
            pltpu.sync_copy(cls_hbm.at[pl.ds(8 * b_id, 8)], cls_buf)
            pltpu.sync_copy(cls_buf.at[pl.ds(0, 1)],
                            out_hbm.at[b_id, pl.ds(0, 1), :])

        @pl.when(q == QUARTERS - 1)
        def _():
            g = (img_v[pl.ds(PER_W - 16, 16)] * (B * N)
                 + (tok_v[pl.ds(PER_W - 16, 16)] + row_off))
            plsc.store_scatter(tidx_v, [jnp.zeros((16,), jnp.int32)], g,
                               mask=lane == 15)
            pltpu.async_copy(
                table_hbm.at[tidx_v.at[pl.ds(0, 1)]], row_buf, sem0).wait()
            pltpu.sync_copy(row_buf, out_hbm.at[b_id, pl.ds(T, 1), :])

    return k(table, img_flat, tok_flat, cls_rows)


def kernel(class_tokens, patch_tokens, src_img, src_tok, W, b):
    Bn = src_img.shape[0]
    cls2 = jnp.transpose(class_tokens, (1, 2, 0, 3)).reshape(Bn, S * H)
    cls_rows = _class_project(cls2, W, b.reshape(1, H))

    table = patch_tokens.reshape(S * B * N, H)
    img_flat = src_img.astype(jnp.int32).reshape(B * T)
    tok_flat = src_tok.astype(jnp.int32).reshape(B * T)

    return _sc_gather(table, img_flat, tok_flat, cls_rows)


# speedup vs baseline: 7.8172x; 1.9207x over previous
import functools

import jax
import jax.numpy as jnp
from jax import lax
from jax.experimental import pallas as pl
from jax.experimental.pallas import tpu as pltpu
from jax.experimental.pallas import tpu_sc as plsc

S, B, N, H, T = 4, 8, 1024, 768, 4096

NC, NS, L = 2, 16, 16
NW = NC * NS
TPW = T // NW
PER_W = TPW * B
CHUNK = 64
NCHUNK = PER_W // CHUNK


def _cls_body(cls_ref, w_ref, b_ref, out_ref):
    out_ref[...] = lax.dot_general(
        cls_ref[...], w_ref[...], (((1,), (1,)), ((), ())),
        preferred_element_type=jnp.float32) + b_ref[...]


def _class_project(cls2, W, b2):
    return pl.pallas_call(
        _cls_body,
        out_shape=jax.ShapeDtypeStruct((B, H), jnp.float32),
    )(cls2, W, b2)


def _sc_gather(table, src_img, src_tok, cls_rows):
    mesh = plsc.VectorSubcoreMesh(
        core_axis_name="c", subcore_axis_name="s",
        num_cores=NC, num_subcores=NS)

    @functools.partial(
        pl.kernel,
        out_type=jax.ShapeDtypeStruct(((T + 1) * B, H), jnp.float32),
        mesh=mesh,
        compiler_params=pltpu.CompilerParams(needs_layout_passes=False),
        scratch_types=[
            pltpu.VMEM((B, TPW), jnp.int32),
            pltpu.VMEM((B, TPW), jnp.int32),
            pltpu.VMEM((PER_W,), jnp.int32),
            pltpu.VMEM((CHUNK, H), jnp.float32),
            pltpu.VMEM((CHUNK, H), jnp.float32),
            pltpu.VMEM((B, H), jnp.float32),
            pltpu.SemaphoreType.DMA,
            pltpu.SemaphoreType.DMA,
        ],
    )
    def k(table_hbm, img_hbm, tok_hbm, cls_hbm, out_hbm,
          img_v, tok_v, idx_v, buf0, buf1, cls_buf, sem0, sem1):
        wid = lax.axis_index("s") * NC + lax.axis_index("c")
        t0 = wid * TPW

        pltpu.sync_copy(img_hbm.at[:, pl.ds(t0, TPW)], img_v)
        pltpu.sync_copy(tok_hbm.at[:, pl.ds(t0, TPW)], tok_v)

        lane = lax.iota(jnp.int32, L)

        def compute_idx(j, carry):
            r = j * L + lane
            bb = lax.bitwise_and(r, B - 1)
            tt = lax.shift_right_logical(r, 3)
            img = plsc.load_gather(img_v, [bb, tt])
            tok = plsc.load_gather(tok_v, [bb, tt])
            idx_v[pl.ds(j * L, L)] = img * (B * N) + (bb * N + tok)
            return carry

        lax.fori_loop(0, PER_W // L, compute_idx, 0)

        bufs = (buf0, buf1)
        sems = (sem0, sem1)
        dst0 = (1 + t0) * B
        prev = pltpu.async_copy(
            table_hbm.at[idx_v.at[pl.ds(0, CHUNK)]], buf0, sem0)
        for c in range(1, NCHUNK):
            cur = pltpu.async_copy(
                table_hbm.at[idx_v.at[pl.ds(c * CHUNK, CHUNK)]],
                bufs[c % 2], sems[c % 2])
            prev.wait()
            pltpu.sync_copy(
                bufs[(c - 1) % 2],
                out_hbm.at[pl.ds(dst0 + (c - 1) * CHUNK, CHUNK)])
            prev = cur
        prev.wait()
        pltpu.sync_copy(
            bufs[(NCHUNK - 1) % 2],
            out_hbm.at[pl.ds(dst0 + (NCHUNK - 1) * CHUNK, CHUNK)])

        @pl.when(wid == 0)
        def _():
            pltpu.sync_copy(cls_hbm, cls_buf)
            pltpu.sync_copy(cls_buf, out_hbm.at[pl.ds(0, B)])

    return k(table, src_img, src_tok, cls_rows)


def kernel(class_tokens, patch_tokens, src_img, src_tok, W, b):
    Bn = src_img.shape[0]
    cls2 = jnp.transpose(class_tokens, (1, 2, 0, 3)).reshape(Bn, S * H)
    cls_rows = _class_project(cls2, W, b.reshape(1, H))

    table = patch_tokens.reshape(S * B * N, H)
    out = _sc_gather(table, src_img.astype(jnp.int32),
                     src_tok.astype(jnp.int32), cls_rows)
    return out.reshape(T + 1, Bn, H).transpose(1, 0, 2)


# speedup vs baseline: 8.0281x; 1.0270x over previous
import functools

import jax
import jax.numpy as jnp
from jax import lax
from jax.experimental import pallas as pl
from jax.experimental.pallas import tpu as pltpu
from jax.experimental.pallas import tpu_sc as plsc

S, B, N, H, T = 4, 8, 1024, 768, 4096

NC, NS, L = 2, 16, 16
NW = NC * NS
TPW = T // NW
PER_W = TPW * B
CHUNK = 32
NCHUNK = PER_W // CHUNK
NBUF = 5


def _cls_body(cls_ref, w_ref, b_ref, out_ref):
    out_ref[...] = lax.dot_general(
        cls_ref[...], w_ref[...], (((1,), (1,)), ((), ())),
        preferred_element_type=jnp.float32) + b_ref[...]


def _class_project(cls2, W, b2):
    return pl.pallas_call(
        _cls_body,
        out_shape=jax.ShapeDtypeStruct((B, H), jnp.float32),
    )(cls2, W, b2)


def _patch_body(big_ref, cls_ref, out_ref, sem):
    del big_ref
    cp = pltpu.make_async_copy(cls_ref, out_ref.at[pl.ds(0, B)], sem)
    cp.start()
    cp.wait()


def _patch_class(big, cls_rows):
    return pl.pallas_call(
        _patch_body,
        in_specs=[
            pl.BlockSpec(memory_space=pl.ANY),
            pl.BlockSpec(memory_space=pl.ANY),
        ],
        out_specs=pl.BlockSpec(memory_space=pl.ANY),
        out_shape=jax.ShapeDtypeStruct(((T + 1) * B, H), jnp.float32),
        scratch_shapes=[pltpu.SemaphoreType.DMA],
        input_output_aliases={0: 0},
    )(big, cls_rows)


def _sc_gather(table, src_img, src_tok):
    mesh = plsc.VectorSubcoreMesh(
        core_axis_name="c", subcore_axis_name="s",
        num_cores=NC, num_subcores=NS)

    @functools.partial(
        pl.kernel,
        out_type=jax.ShapeDtypeStruct(((T + 1) * B, H), jnp.float32),
        mesh=mesh,
        compiler_params=pltpu.CompilerParams(needs_layout_passes=False),
        scratch_types=[
            pltpu.VMEM((B, TPW), jnp.int32),
            pltpu.VMEM((B, TPW), jnp.int32),
            pltpu.VMEM((PER_W,), jnp.int32),
            *[pltpu.VMEM((CHUNK, H), jnp.float32) for _ in range(NBUF)],
            *[pltpu.SemaphoreType.DMA for _ in range(NBUF)],
            *[pltpu.SemaphoreType.DMA for _ in range(NBUF)],
        ],
    )
    def k(table_hbm, img_hbm, tok_hbm, out_hbm,
          img_v, tok_v, idx_v, *bufs_and_sems):
        bufs = bufs_and_sems[:NBUF]
        gsems = bufs_and_sems[NBUF:2 * NBUF]
        wsems = bufs_and_sems[2 * NBUF:3 * NBUF]
        wid = lax.axis_index("s") * NC + lax.axis_index("c")
        t0 = wid * TPW

        pltpu.sync_copy(img_hbm.at[:, pl.ds(t0, TPW)], img_v)
        pltpu.sync_copy(tok_hbm.at[:, pl.ds(t0, TPW)], tok_v)

        lane = lax.iota(jnp.int32, L)

        def compute_idx(j, carry):
            r = j * L + lane
            bb = lax.bitwise_and(r, B - 1)
            tt = lax.shift_right_logical(r, 3)
            img = plsc.load_gather(img_v, [bb, tt])
            tok = plsc.load_gather(tok_v, [bb, tt])
            idx_v[pl.ds(j * L, L)] = img * (B * N) + (bb * N + tok)
            return carry

        lax.fori_loop(0, PER_W // L, compute_idx, 0)

        dst0 = (1 + t0) * B
        gathers = [None] * NCHUNK
        writes = [None] * NCHUNK

        def issue_gather(c):
            gathers[c] = pltpu.async_copy(
                table_hbm.at[idx_v.at[pl.ds(c * CHUNK, CHUNK)]],
                bufs[c % NBUF], gsems[c % NBUF])

        def issue_write(c):
            writes[c] = pltpu.async_copy(
                bufs[c % NBUF],
                out_hbm.at[pl.ds(dst0 + c * CHUNK, CHUNK)],
                wsems[c % NBUF])

        for c in range(NCHUNK + NBUF - 1):
            if c < NCHUNK:
                if c >= NBUF:
                    writes[c - NBUF].wait()
                issue_gather(c)
            d = c - (NBUF - 1)
            if 0 <= d < NCHUNK:
                gathers[d].wait()
                issue_write(d)
        for c in range(NCHUNK - NBUF, NCHUNK):
            writes[c].wait()

    return k(table, src_img, src_tok)


def kernel(class_tokens, patch_tokens, src_img, src_tok, W, b):
    Bn = src_img.shape[0]
    cls2 = jnp.transpose(class_tokens, (1, 2, 0, 3)).reshape(Bn, S * H)
    cls_rows = _class_project(cls2, W, b.reshape(1, H))

    table = patch_tokens.reshape(S * B * N, H)
    out = _sc_gather(table, src_img.astype(jnp.int32),
                     src_tok.astype(jnp.int32))
    out = _patch_class(out, cls_rows)
    return out.reshape(T + 1, Bn, H).transpose(1, 0, 2)
